# baseline (speedup 1.0000x reference)
import functools

import jax
import jax.numpy as jnp
from jax import lax
from jax.experimental import pallas as pl
from jax.experimental.pallas import tpu as pltpu
from jax.experimental.pallas import tpu_sc as plsc

_F = 26
_V = 100000
_D = 32
_B = 16384
_NC = 13
_R = _F * _D + _NC

_NWORKERS = 32
_CHUNK = 4096
_NCHUNK = _B // _CHUNK


def _emb_body(xt_hbm, tab_hbm, out_hbm, trow, xrow, ob0, ob1, sem, sw0, sw1):
    cid = lax.axis_index("c")
    sid = lax.axis_index("s")
    wid = sid * 2 + cid
    start = wid * 26 + jnp.minimum(wid, 13)
    cnt = 26 + jnp.where(wid < 13, 1, 0)
    obufs = (ob0, ob1)
    swsems = (sw0, sw1)

    def row_body(i, carry):
        r = start + i
        is_emb = r < _F * _D
        f = r >> 5
        xid = jnp.where(is_emb, f, r - _F * _D + _F)
        need_x = jnp.logical_or(
            jnp.logical_or(i == 0, lax.rem(r, _D) == 0),
            jnp.logical_not(is_emb),
        )

        @pl.when(need_x)
        def _():
            pltpu.async_copy(xt_hbm.at[xid], xrow, sem).wait()

        @pl.when(is_emb)
        def _():
            e = r & (_D - 1)
            pltpu.async_copy(tab_hbm.at[f, e], trow, sem).wait()

        for c2 in range(_NCHUNK):
            ob = obufs[c2 % 2]
            sw = swsems[c2 % 2]
            drain = pltpu.make_async_copy(
                ob, out_hbm.at[r, pl.ds(c2 * _CHUNK, _CHUNK)], sw
            )
            if c2 >= 2:
                drain.wait()
            else:
                @pl.when(i > 0)
                def _(d=drain):
                    d.wait()

            base = c2 * _CHUNK

            @pl.when(is_emb)
            def _(ob=ob, base=base):
                def gs(j, c):
                    v = xrow[pl.ds(base + j * 16, 16)]
                    ob[pl.ds(j * 16, 16)] = plsc.load_gather(trow, [v])
                    return c

                lax.fori_loop(0, _CHUNK // 16, gs, 0, unroll=8)

            @pl.when(jnp.logical_not(is_emb))
            def _(ob=ob, base=base):
                def cs(j, c):
                    v = xrow[pl.ds(base + j * 16, 16)]
                    ob[pl.ds(j * 16, 16)] = v.astype(jnp.float32)
                    return c

                lax.fori_loop(0, _CHUNK // 16, cs, 0, unroll=8)

            pltpu.async_copy(
                ob, out_hbm.at[r, pl.ds(c2 * _CHUNK, _CHUNK)], sw
            )
        return carry

    lax.fori_loop(0, cnt, row_body, 0)

    r_last = start + cnt - 1
    for c2 in (2, 3):
        pltpu.make_async_copy(
            obufs[c2 % 2],
            out_hbm.at[r_last, pl.ds(c2 * _CHUNK, _CHUNK)],
            swsems[c2 % 2],
        ).wait()


@jax.jit
def _emb_call(xt, tabt):
    mesh = plsc.VectorSubcoreMesh(core_axis_name="c", subcore_axis_name="s")
    run = functools.partial(
        pl.kernel,
        mesh=mesh,
        out_type=jax.ShapeDtypeStruct((_R, _B), jnp.float32),
        scratch_types=[
            pltpu.VMEM((_V,), jnp.float32),
            pltpu.VMEM((_B,), jnp.int32),
            pltpu.VMEM((_CHUNK,), jnp.float32),
            pltpu.VMEM((_CHUNK,), jnp.float32),
            pltpu.SemaphoreType.DMA,
            pltpu.SemaphoreType.DMA,
            pltpu.SemaphoreType.DMA,
        ],
        compiler_params=pltpu.CompilerParams(
            use_tc_tiling_on_sc=False, needs_layout_passes=False
        ),
    )(_emb_body)
    return run(xt, tabt)


def kernel(X, tables):
    xt = X.astype(jnp.int32).T
    tabt = tables.transpose(0, 2, 1)
    return _emb_call(xt, tabt).T


# speedup vs baseline: 1.0842x; 1.0842x over previous
import functools

import jax
import jax.numpy as jnp
from jax import lax
from jax.experimental import pallas as pl
from jax.experimental.pallas import tpu as pltpu
from jax.experimental.pallas import tpu_sc as plsc

_F = 26
_V = 100000
_D = 32
_B = 16384
_NC = 13
_R = _F * _D + _NC

_NWORKERS = 32
_CHUNK = 4096
_NCHUNK = _B // _CHUNK


def _emb_body(xt_hbm, tab_hbm, out_hbm, trow, xrow, ob0, ob1, sem, sw0, sw1):
    cid = lax.axis_index("c")
    sid = lax.axis_index("s")
    wid = sid * 2 + cid
    start = wid * 26 + jnp.minimum(wid, 13)
    cnt = 26 + jnp.where(wid < 13, 1, 0)
    obufs = (ob0, ob1)
    swsems = (sw0, sw1)

    def row_body(i, carry):
        r = start + i
        is_emb = r < _F * _D
        f = r >> 5
        xid = jnp.where(is_emb, f, r - _F * _D + _F)
        need_x = jnp.logical_or(
            jnp.logical_or(i == 0, lax.rem(r, _D) == 0),
            jnp.logical_not(is_emb),
        )

        @pl.when(need_x)
        def _():
            pltpu.async_copy(xt_hbm.at[xid], xrow, sem).wait()

        @pl.when(is_emb)
        def _():
            e = r & (_D - 1)
            pltpu.async_copy(tab_hbm.at[f, e], trow, sem).wait()

        for c2 in range(_NCHUNK):
            ob = obufs[c2 % 2]
            sw = swsems[c2 % 2]
            drain = pltpu.make_async_copy(
                ob, out_hbm.at[r, pl.ds(c2 * _CHUNK, _CHUNK)], sw
            )
            if c2 >= 2:
                drain.wait()
            else:
                @pl.when(i > 0)
                def _(d=drain):
                    d.wait()

            base = c2 * _CHUNK

            @pl.when(is_emb)
            def _(ob=ob, base=base):
                def gs(j, c):
                    v = xrow[pl.ds(base + j * 16, 16)]
                    ob[pl.ds(j * 16, 16)] = plsc.load_gather(trow, [v])
                    return c

                lax.fori_loop(0, _CHUNK // 16, gs, 0)

            @pl.when(jnp.logical_not(is_emb))
            def _(ob=ob, base=base):
                def cs(j, c):
                    v = xrow[pl.ds(base + j * 16, 16)]
                    ob[pl.ds(j * 16, 16)] = v.astype(jnp.float32)
                    return c

                lax.fori_loop(0, _CHUNK // 16, cs, 0)

            pltpu.async_copy(
                ob, out_hbm.at[r, pl.ds(c2 * _CHUNK, _CHUNK)], sw
            )
        return carry

    lax.fori_loop(0, cnt, row_body, 0)

    r_last = start + cnt - 1
    for c2 in (2, 3):
        pltpu.make_async_copy(
            obufs[c2 % 2],
            out_hbm.at[r_last, pl.ds(c2 * _CHUNK, _CHUNK)],
            swsems[c2 % 2],
        ).wait()


@jax.jit
def _emb_call(xt, tabt):
    mesh = plsc.VectorSubcoreMesh(core_axis_name="c", subcore_axis_name="s")
    run = functools.partial(
        pl.kernel,
        mesh=mesh,
        out_type=jax.ShapeDtypeStruct((_R, _B), jnp.float32),
        scratch_types=[
            pltpu.VMEM((_V,), jnp.float32),
            pltpu.VMEM((_B,), jnp.int32),
            pltpu.VMEM((_CHUNK,), jnp.float32),
            pltpu.VMEM((_CHUNK,), jnp.float32),
            pltpu.SemaphoreType.DMA,
            pltpu.SemaphoreType.DMA,
            pltpu.SemaphoreType.DMA,
        ],
        compiler_params=pltpu.CompilerParams(
            use_tc_tiling_on_sc=False, needs_layout_passes=False
        ),
    )(_emb_body)
    return run(xt, tabt)


def kernel(X, tables):
    xt = X.astype(jnp.int32).T
    tabt = tables.transpose(0, 2, 1)
    return _emb_call(xt, tabt).T
